# Initial kernel scaffold; baseline (speedup 1.0000x reference)
import jax
import jax.numpy as jnp
from jax.experimental import pallas as pl


def kernel(x, edge_index, W1, b1, g1, be1, W2, b2, g2, be2):
    raise NotImplementedError("write your pallas kernel here")


# speedup vs baseline: 3.6308x; 3.6308x over previous
import functools

import jax
import jax.numpy as jnp
from jax import lax
from jax.experimental import pallas as pl
from jax.experimental.pallas import tpu as pltpu, tpu_sc as plsc

N = 10000
E = 320000
D = 128

NC = 2
NS = 16
NW = NC * NS
CL = 128
CPT = 80
TOT_CHUNKS = NW * CPT
E_PAD = TOT_CHUNKS * CL
N_R = 10112
RPT = N_R // NS
GRID = N_R // 128

_mesh = plsc.VectorSubcoreMesh(
    core_axis_name="c", subcore_axis_name="s", num_cores=NC, num_subcores=NS)

_ZCHUNKS = [(0, 128), (128, 128), (256, 128), (384, 128), (512, 120)]


ND = 10240
NDT = ND // NS


def _deg_body(src_hbm, dst_hbm, out_hbm, idx_v, dsrc_v, ddst_v, buf_v, res_v, grid_sh):
  core = lax.axis_index("c")
  sub = lax.axis_index("s")
  wid = core * NS + sub

  zeros16 = jnp.zeros((16,), jnp.float32)
  ones16 = jnp.ones((16,), jnp.float32)

  def _zero(j, carry):
    dsrc_v[pl.ds(j * 16, 16)] = zeros16
    ddst_v[pl.ds(j * 16, 16)] = zeros16
    return carry

  lax.fori_loop(0, ND // 16, _zero, 0)

  pltpu.sync_copy(src_hbm.at[pl.ds(wid * CPT, CPT)], idx_v.at[0])
  pltpu.sync_copy(dst_hbm.at[pl.ds(wid * CPT, CPT)], idx_v.at[1])

  def _step(c, carry):
    for k in range(CL // 16):
      i_src = idx_v[0, c, pl.ds(k * 16, 16)]
      plsc.addupdate_scatter(dsrc_v, [i_src], ones16)
      i_dst = idx_v[1, c, pl.ds(k * 16, 16)]
      plsc.addupdate_scatter(ddst_v, [i_dst], ones16)
    return carry

  lax.fori_loop(0, CPT, _step, 0)

  pltpu.sync_copy(dsrc_v, grid_sh.at[sub, 0])
  pltpu.sync_copy(ddst_v, grid_sh.at[sub, 1])
  plsc.subcore_barrier()

  for d in range(2):
    for t in range(NS):
      pltpu.sync_copy(grid_sh.at[t, d, pl.ds(sub * NDT, NDT)], buf_v.at[t])

    def _red(j, carry):
      acc = jnp.zeros((16,), jnp.float32)
      for t in range(NS):
        acc = acc + buf_v[t, pl.ds(j * 16, 16)]
      res_v[pl.ds(j * 16, 16)] = acc
      return carry

    lax.fori_loop(0, NDT // 16, _red, 0)
    pltpu.sync_copy(res_v, out_hbm.at[core, d, pl.ds(sub * NDT, NDT)])


_deg = pl.kernel(
    _deg_body,
    out_type=jax.ShapeDtypeStruct((NC, 2, ND), jnp.float32),
    mesh=_mesh,
    scratch_types=[
        pltpu.VMEM((2, CPT, CL), jnp.int32),
        pltpu.VMEM((ND,), jnp.float32),
        pltpu.VMEM((ND,), jnp.float32),
        pltpu.VMEM((NS, NDT), jnp.float32),
        pltpu.VMEM((NDT,), jnp.float32),
        pltpu.VMEM_SHARED((NS, 2, ND), jnp.float32),
    ],
    compiler_params=pltpu.CompilerParams(needs_layout_passes=False),
)


GROUPS = 2
GC = CPT // GROUPS


def _agg_body(h_hbm, src_hbm, dst_hbm, out_hbm,
              idx_v, rows0, rows1, sem0, sem1, agg_sh):
  core = lax.axis_index("c")
  sub = lax.axis_index("s")
  wid = core * NS + sub

  def _fill(r, carry):
    for cgrp in range(D // 16):
      rows0[r, pl.ds(cgrp * 16, 16)] = jnp.zeros((16,), jnp.float32)
    return carry

  lax.fori_loop(0, CL, _fill, 0)

  for off, ln in _ZCHUNKS:
    pltpu.sync_copy(rows0.at[pl.ds(0, ln)], agg_sh.at[pl.ds(sub * RPT + off, ln)])
  plsc.subcore_barrier()

  def _group(gi, carry):
    base = wid * CPT + gi * GC
    pltpu.sync_copy(src_hbm.at[pl.ds(base, GC)], idx_v.at[0])
    pltpu.sync_copy(dst_hbm.at[pl.ds(base, GC)], idx_v.at[1])
    pltpu.async_copy(h_hbm.at[idx_v.at[0, 0]], rows0, sem0)

    def _step(g, carry2):
      c0 = 2 * g
      c1 = c0 + 1
      c2 = c0 + 2
      pltpu.async_copy(h_hbm.at[idx_v.at[0, c1]], rows1, sem1)
      pltpu.make_async_copy(h_hbm.at[idx_v.at[0, c0]], rows0, sem0).wait()
      pltpu.sync_copy(rows0, agg_sh.at[idx_v.at[1, c0]], add=True)

      @pl.when(c2 < GC)
      def _():
        pltpu.async_copy(h_hbm.at[idx_v.at[0, c2]], rows0, sem0)

      pltpu.make_async_copy(h_hbm.at[idx_v.at[0, c1]], rows1, sem1).wait()
      pltpu.sync_copy(rows1, agg_sh.at[idx_v.at[1, c1]], add=True)
      return carry2

    lax.fori_loop(0, GC // 2, _step, 0)
    return carry

  lax.fori_loop(0, GROUPS, _group, 0)
  plsc.subcore_barrier()

  pltpu.sync_copy(agg_sh.at[pl.ds(sub * RPT, RPT)],
                  out_hbm.at[core, pl.ds(sub * RPT, RPT)])


_agg = pl.kernel(
    _agg_body,
    out_type=jax.ShapeDtypeStruct((NC, N_R, D), jnp.float32),
    mesh=_mesh,
    scratch_types=[
        pltpu.VMEM((2, GC, CL), jnp.int32),
        pltpu.VMEM((CL, D), jnp.float32),
        pltpu.VMEM((CL, D), jnp.float32),
        pltpu.SemaphoreType.DMA,
        pltpu.SemaphoreType.DMA,
        pltpu.VMEM_SHARED((N_R, D), jnp.float32),
    ],
)


def _prescale_body(x_ref, deg_ref, xs_ref, nsrc_ref, ndst_ref):
  d = deg_ref[...]
  dsum = d[0] + d[1]
  dsrc = dsum[0]
  ddst = dsum[1]
  ns = jnp.where(dsrc > 0, lax.rsqrt(dsrc), 0.0)
  nd = jnp.where(ddst > 0, lax.rsqrt(ddst), 0.0)
  xs_ref[...] = x_ref[...] * ns
  nsrc_ref[...] = ns
  ndst_ref[...] = nd


def _prescale(x_pad, degs):
  return pl.pallas_call(
      _prescale_body,
      grid=(GRID,),
      in_specs=[
          pl.BlockSpec((128, D), lambda i: (i, 0)),
          pl.BlockSpec((NC, 2, 128, 1), lambda i: (0, 0, i, 0)),
      ],
      out_specs=[
          pl.BlockSpec((128, D), lambda i: (i, 0)),
          pl.BlockSpec((128, 1), lambda i: (i, 0)),
          pl.BlockSpec((128, 1), lambda i: (i, 0)),
      ],
      out_shape=[
          jax.ShapeDtypeStruct((N_R, D), jnp.float32),
          jax.ShapeDtypeStruct((N_R, 1), jnp.float32),
          jax.ShapeDtypeStruct((N_R, 1), jnp.float32),
      ],
  )(x_pad, degs)


def _mm_body(p_ref, nd_ref, w_ref, b_ref, hlin_ref, stats_ref):
  i = pl.program_id(0)
  agg = (p_ref[0] + p_ref[1]) * nd_ref[...]
  h = jnp.dot(agg, w_ref[...], preferred_element_type=jnp.float32) + b_ref[...]
  hlin_ref[...] = h
  rows = i * 128 + lax.broadcasted_iota(jnp.int32, (128, 1), 0)
  hm = jnp.where(rows < N, h, 0.0)
  s = jnp.sum(hm, axis=0, keepdims=True)
  s2 = jnp.sum(hm * hm, axis=0, keepdims=True)
  stats_ref[...] = jnp.concatenate([s, s2], axis=1).reshape(1, 1, 2 * D)


def _mm(parts, ndst, W, b):
  return pl.pallas_call(
      _mm_body,
      grid=(GRID,),
      in_specs=[
          pl.BlockSpec((NC, 128, D), lambda i: (0, i, 0)),
          pl.BlockSpec((128, 1), lambda i: (i, 0)),
          pl.BlockSpec((D, D), lambda i: (0, 0)),
          pl.BlockSpec((1, D), lambda i: (0, 0)),
      ],
      out_specs=[
          pl.BlockSpec((128, D), lambda i: (i, 0)),
          pl.BlockSpec((1, 1, 2 * D), lambda i: (i, 0, 0)),
      ],
      out_shape=[
          jax.ShapeDtypeStruct((N_R, D), jnp.float32),
          jax.ShapeDtypeStruct((GRID, 1, 2 * D), jnp.float32),
      ],
  )(parts, ndst, W, b.reshape(1, D))


def _bn_body(h_ref, st_ref, g_ref, be_ref, ns_ref, o_ref, *, scale):
  st = jnp.sum(st_ref[...], axis=0)
  s = st[:, :D]
  s2 = st[:, D:]
  mean = s * (1.0 / N)
  var = s2 * (1.0 / N) - mean * mean
  inv = lax.rsqrt(var + 1e-5)
  h = (h_ref[...] - mean) * inv * g_ref[...] + be_ref[...]
  h = jnp.maximum(h, 0.0)
  if scale:
    h = h * ns_ref[...]
  o_ref[...] = h


def _bn(hlin, stats, g, be, nsrc, scale):
  return pl.pallas_call(
      functools.partial(_bn_body, scale=scale),
      grid=(GRID,),
      in_specs=[
          pl.BlockSpec((128, D), lambda i: (i, 0)),
          pl.BlockSpec((GRID, 1, 2 * D), lambda i: (0, 0, 0)),
          pl.BlockSpec((1, D), lambda i: (0, 0)),
          pl.BlockSpec((1, D), lambda i: (0, 0)),
          pl.BlockSpec((128, 1), lambda i: (i, 0)),
      ],
      out_specs=pl.BlockSpec((128, D), lambda i: (i, 0)),
      out_shape=jax.ShapeDtypeStruct((N_R, D), jnp.float32),
  )(hlin, stats, g.reshape(1, D), be.reshape(1, D), nsrc)


def kernel(x, edge_index, W1, b1, g1, be1, W2, b2, g2, be2):
  src = edge_index[0]
  dst = edge_index[1]
  pad = jnp.full((E_PAD - E,), N, dtype=jnp.int32)
  src_rows = jnp.concatenate([src, pad]).reshape(TOT_CHUNKS, CL)
  dst_rows = jnp.concatenate([dst, pad]).reshape(TOT_CHUNKS, CL)
  x_pad = jnp.concatenate([x, jnp.zeros((N_R - N, D), jnp.float32)], axis=0)

  degs = _deg(src_rows, dst_rows).reshape(NC, 2, ND, 1)
  xs, nsrc, ndst = _prescale(x_pad, degs)

  parts1 = _agg(xs, src_rows, dst_rows)
  hlin1, st1 = _mm(parts1, ndst, W1, b1)
  h1s = _bn(hlin1, st1, g1, be1, nsrc, scale=True)

  parts2 = _agg(h1s, src_rows, dst_rows)
  hlin2, st2 = _mm(parts2, ndst, W2, b2)
  h2 = _bn(hlin2, st2, g2, be2, nsrc, scale=False)
  return h2[:N]


# speedup vs baseline: 3.7325x; 1.0280x over previous
import functools

import jax
import jax.numpy as jnp
from jax import lax
from jax.experimental import pallas as pl
from jax.experimental.pallas import tpu as pltpu, tpu_sc as plsc

N = 10000
E = 320000
D = 128

NC = 2
NS = 16
NW = NC * NS
CL = 128
CPT = 80
TOT_CHUNKS = NW * CPT
E_PAD = TOT_CHUNKS * CL
N_R = 10112
RPT = N_R // NS
GRID = N_R // 128

_mesh = plsc.VectorSubcoreMesh(
    core_axis_name="c", subcore_axis_name="s", num_cores=NC, num_subcores=NS)

_ZCHUNKS = [(0, 128), (128, 128), (256, 128), (384, 128), (512, 120)]


ND = 10240
NDT = ND // NS


def _deg_body(src_hbm, dst_hbm, out_hbm, idx_v, dsrc_v, ddst_v, buf_v, res_v, grid_sh):
  core = lax.axis_index("c")
  sub = lax.axis_index("s")
  wid = core * NS + sub

  zeros16 = jnp.zeros((16,), jnp.float32)
  ones16 = jnp.ones((16,), jnp.float32)

  def _zero(j, carry):
    dsrc_v[pl.ds(j * 16, 16)] = zeros16
    ddst_v[pl.ds(j * 16, 16)] = zeros16
    return carry

  lax.fori_loop(0, ND // 16, _zero, 0)

  pltpu.sync_copy(src_hbm.at[pl.ds(wid * CPT, CPT)], idx_v.at[0])
  pltpu.sync_copy(dst_hbm.at[pl.ds(wid * CPT, CPT)], idx_v.at[1])

  def _step(c, carry):
    for k in range(CL // 16):
      i_src = idx_v[0, c, pl.ds(k * 16, 16)]
      plsc.addupdate_scatter(dsrc_v, [i_src], ones16)
      i_dst = idx_v[1, c, pl.ds(k * 16, 16)]
      plsc.addupdate_scatter(ddst_v, [i_dst], ones16)
    return carry

  lax.fori_loop(0, CPT, _step, 0)

  pltpu.sync_copy(dsrc_v, grid_sh.at[sub, 0])
  pltpu.sync_copy(ddst_v, grid_sh.at[sub, 1])
  plsc.subcore_barrier()

  for d in range(2):
    for t in range(NS):
      pltpu.sync_copy(grid_sh.at[t, d, pl.ds(sub * NDT, NDT)], buf_v.at[t])

    def _red(j, carry):
      acc = jnp.zeros((16,), jnp.float32)
      for t in range(NS):
        acc = acc + buf_v[t, pl.ds(j * 16, 16)]
      res_v[pl.ds(j * 16, 16)] = acc
      return carry

    lax.fori_loop(0, NDT // 16, _red, 0)
    pltpu.sync_copy(res_v, out_hbm.at[core, d, pl.ds(sub * NDT, NDT)])


_deg = pl.kernel(
    _deg_body,
    out_type=jax.ShapeDtypeStruct((NC, 2, ND), jnp.float32),
    mesh=_mesh,
    scratch_types=[
        pltpu.VMEM((2, CPT, CL), jnp.int32),
        pltpu.VMEM((ND,), jnp.float32),
        pltpu.VMEM((ND,), jnp.float32),
        pltpu.VMEM((NS, NDT), jnp.float32),
        pltpu.VMEM((NDT,), jnp.float32),
        pltpu.VMEM_SHARED((NS, 2, ND), jnp.float32),
    ],
    compiler_params=pltpu.CompilerParams(needs_layout_passes=False),
)


CPT0 = 128
CPT1 = 32
GC = 32
assert NS * (CPT0 + CPT1) == TOT_CHUNKS


def _agg_body(h_hbm, src_hbm, dst_hbm, out_hbm,
              idx_v, rows0, rows1, sem0, sem1, agg_sh):
  core = lax.axis_index("c")
  sub = lax.axis_index("s")

  def _fill(r, carry):
    for cgrp in range(D // 16):
      rows0[r, pl.ds(cgrp * 16, 16)] = jnp.zeros((16,), jnp.float32)
    return carry

  lax.fori_loop(0, CL, _fill, 0)

  for off, ln in _ZCHUNKS:
    pltpu.sync_copy(rows0.at[pl.ds(0, ln)], agg_sh.at[pl.ds(sub * RPT + off, ln)])
  plsc.subcore_barrier()

  def _run_group(base):
    pltpu.sync_copy(src_hbm.at[pl.ds(base, GC)], idx_v.at[0])
    pltpu.sync_copy(dst_hbm.at[pl.ds(base, GC)], idx_v.at[1])
    pltpu.async_copy(h_hbm.at[idx_v.at[0, 0]], rows0, sem0)

    def _step(g, carry2):
      c0 = 2 * g
      c1 = c0 + 1
      c2 = c0 + 2
      pltpu.async_copy(h_hbm.at[idx_v.at[0, c1]], rows1, sem1)
      pltpu.make_async_copy(h_hbm.at[idx_v.at[0, c0]], rows0, sem0).wait()
      pltpu.sync_copy(rows0, agg_sh.at[idx_v.at[1, c0]], add=True)

      @pl.when(c2 < GC)
      def _():
        pltpu.async_copy(h_hbm.at[idx_v.at[0, c2]], rows0, sem0)

      pltpu.make_async_copy(h_hbm.at[idx_v.at[0, c1]], rows1, sem1).wait()
      pltpu.sync_copy(rows1, agg_sh.at[idx_v.at[1, c1]], add=True)
      return carry2

    lax.fori_loop(0, GC // 2, _step, 0)

  @pl.when(core == 0)
  def _():
    for gi in range(CPT0 // GC):
      _run_group(sub * CPT0 + gi * GC)

  @pl.when(core == 1)
  def _():
    for gi in range(CPT1 // GC):
      _run_group(NS * CPT0 + sub * CPT1 + gi * GC)

  plsc.subcore_barrier()

  pltpu.sync_copy(agg_sh.at[pl.ds(sub * RPT, RPT)],
                  out_hbm.at[core, pl.ds(sub * RPT, RPT)])


_agg = pl.kernel(
    _agg_body,
    out_type=jax.ShapeDtypeStruct((NC, N_R, D), jnp.float32),
    mesh=_mesh,
    scratch_types=[
        pltpu.VMEM((2, GC, CL), jnp.int32),
        pltpu.VMEM((CL, D), jnp.float32),
        pltpu.VMEM((CL, D), jnp.float32),
        pltpu.SemaphoreType.DMA,
        pltpu.SemaphoreType.DMA,
        pltpu.VMEM_SHARED((N_R, D), jnp.float32),
    ],
)


def _prescale_body(x_ref, deg_ref, xs_ref, nsrc_ref, ndst_ref):
  d = deg_ref[...]
  dsum = d[0] + d[1]
  dsrc = dsum[0]
  ddst = dsum[1]
  ns = jnp.where(dsrc > 0, lax.rsqrt(dsrc), 0.0)
  nd = jnp.where(ddst > 0, lax.rsqrt(ddst), 0.0)
  xs_ref[...] = x_ref[...] * ns
  nsrc_ref[...] = ns
  ndst_ref[...] = nd


def _prescale(x_pad, degs):
  return pl.pallas_call(
      _prescale_body,
      grid=(GRID,),
      in_specs=[
          pl.BlockSpec((128, D), lambda i: (i, 0)),
          pl.BlockSpec((NC, 2, 128, 1), lambda i: (0, 0, i, 0)),
      ],
      out_specs=[
          pl.BlockSpec((128, D), lambda i: (i, 0)),
          pl.BlockSpec((128, 1), lambda i: (i, 0)),
          pl.BlockSpec((128, 1), lambda i: (i, 0)),
      ],
      out_shape=[
          jax.ShapeDtypeStruct((N_R, D), jnp.float32),
          jax.ShapeDtypeStruct((N_R, 1), jnp.float32),
          jax.ShapeDtypeStruct((N_R, 1), jnp.float32),
      ],
  )(x_pad, degs)


def _mm_body(p_ref, nd_ref, w_ref, b_ref, hlin_ref, stats_ref):
  i = pl.program_id(0)
  agg = (p_ref[0] + p_ref[1]) * nd_ref[...]
  h = jnp.dot(agg, w_ref[...], preferred_element_type=jnp.float32) + b_ref[...]
  hlin_ref[...] = h
  rows = i * 128 + lax.broadcasted_iota(jnp.int32, (128, 1), 0)
  hm = jnp.where(rows < N, h, 0.0)
  s = jnp.sum(hm, axis=0, keepdims=True)
  s2 = jnp.sum(hm * hm, axis=0, keepdims=True)
  stats_ref[...] = jnp.concatenate([s, s2], axis=1).reshape(1, 1, 2 * D)


def _mm(parts, ndst, W, b):
  return pl.pallas_call(
      _mm_body,
      grid=(GRID,),
      in_specs=[
          pl.BlockSpec((NC, 128, D), lambda i: (0, i, 0)),
          pl.BlockSpec((128, 1), lambda i: (i, 0)),
          pl.BlockSpec((D, D), lambda i: (0, 0)),
          pl.BlockSpec((1, D), lambda i: (0, 0)),
      ],
      out_specs=[
          pl.BlockSpec((128, D), lambda i: (i, 0)),
          pl.BlockSpec((1, 1, 2 * D), lambda i: (i, 0, 0)),
      ],
      out_shape=[
          jax.ShapeDtypeStruct((N_R, D), jnp.float32),
          jax.ShapeDtypeStruct((GRID, 1, 2 * D), jnp.float32),
      ],
  )(parts, ndst, W, b.reshape(1, D))


def _bn_body(h_ref, st_ref, g_ref, be_ref, ns_ref, o_ref, *, scale):
  st = jnp.sum(st_ref[...], axis=0)
  s = st[:, :D]
  s2 = st[:, D:]
  mean = s * (1.0 / N)
  var = s2 * (1.0 / N) - mean * mean
  inv = lax.rsqrt(var + 1e-5)
  h = (h_ref[...] - mean) * inv * g_ref[...] + be_ref[...]
  h = jnp.maximum(h, 0.0)
  if scale:
    h = h * ns_ref[...]
  o_ref[...] = h


def _bn(hlin, stats, g, be, nsrc, scale):
  return pl.pallas_call(
      functools.partial(_bn_body, scale=scale),
      grid=(GRID,),
      in_specs=[
          pl.BlockSpec((128, D), lambda i: (i, 0)),
          pl.BlockSpec((GRID, 1, 2 * D), lambda i: (0, 0, 0)),
          pl.BlockSpec((1, D), lambda i: (0, 0)),
          pl.BlockSpec((1, D), lambda i: (0, 0)),
          pl.BlockSpec((128, 1), lambda i: (i, 0)),
      ],
      out_specs=pl.BlockSpec((128, D), lambda i: (i, 0)),
      out_shape=jax.ShapeDtypeStruct((N_R, D), jnp.float32),
  )(hlin, stats, g.reshape(1, D), be.reshape(1, D), nsrc)


def kernel(x, edge_index, W1, b1, g1, be1, W2, b2, g2, be2):
  src = edge_index[0]
  dst = edge_index[1]
  pad = jnp.full((E_PAD - E,), N, dtype=jnp.int32)
  src_rows = jnp.concatenate([src, pad]).reshape(TOT_CHUNKS, CL)
  dst_rows = jnp.concatenate([dst, pad]).reshape(TOT_CHUNKS, CL)
  x_pad = jnp.concatenate([x, jnp.zeros((N_R - N, D), jnp.float32)], axis=0)

  degs = _deg(src_rows, dst_rows).reshape(NC, 2, ND, 1)
  xs, nsrc, ndst = _prescale(x_pad, degs)

  parts1 = _agg(xs, src_rows, dst_rows)
  hlin1, st1 = _mm(parts1, ndst, W1, b1)
  h1s = _bn(hlin1, st1, g1, be1, nsrc, scale=True)

  parts2 = _agg(h1s, src_rows, dst_rows)
  hlin2, st2 = _mm(parts2, ndst, W2, b2)
  h2 = _bn(hlin2, st2, g2, be2, nsrc, scale=False)
  return h2[:N]
